# baseline (speedup 1.0000x reference)
import functools

import jax
import jax.numpy as jnp
from jax import lax
from jax.experimental import pallas as pl
from jax.experimental.pallas import tpu as pltpu
from jax.experimental.pallas import tpu_sc as plsc

_NC = 2
_NS = 16
_B = 80


def _idx_body(src_ref, et_ref, o_ref, *, n_nodes):
    o_ref[...] = et_ref[...] * n_nodes + src_ref[...]


def _flat_index(src, et, n_nodes):
    e = src.shape[0]
    src2 = src.reshape(e // 128, 128)
    et2 = et.reshape(e // 128, 128)
    rows = e // 128
    out = pl.pallas_call(
        functools.partial(_idx_body, n_nodes=n_nodes),
        out_shape=jax.ShapeDtypeStruct((rows, 128), jnp.int32),
    )(src2, et2)
    return out.reshape(-1)


def _mm_plain_body(x_ref, w_ref, y_ref):
    y_ref[...] = jnp.dot(x_ref[...], w_ref[0], preferred_element_type=jnp.float32)


def _rel_transform(x, Wr, bn):
    n, d = x.shape
    r, _, do = Wr.shape
    nblk = n // bn
    return pl.pallas_call(
        _mm_plain_body,
        grid=(nblk, r),
        in_specs=[
            pl.BlockSpec((bn, d), lambda i, j: (i, 0)),
            pl.BlockSpec((1, d, do), lambda i, j: (j, 0, 0)),
        ],
        out_specs=pl.BlockSpec((bn, do), lambda i, j: (j * nblk + i, 0)),
        out_shape=jax.ShapeDtypeStruct((r * n, do), jnp.float32),
    )(x, Wr)


def _mm_fused_body(p_ref, w_ref, y_ref):
    h = jnp.maximum(p_ref[0] + p_ref[1], 0.0)
    y_ref[...] = jnp.dot(h, w_ref[0], preferred_element_type=jnp.float32)


def _rel_transform_fused(p, Wr, bn):
    _, n, d = p.shape
    r, _, do = Wr.shape
    nblk = n // bn
    return pl.pallas_call(
        _mm_fused_body,
        grid=(nblk, r),
        in_specs=[
            pl.BlockSpec((2, bn, d), lambda i, j: (0, i, 0)),
            pl.BlockSpec((1, d, do), lambda i, j: (j, 0, 0)),
        ],
        out_specs=pl.BlockSpec((bn, do), lambda i, j: (j * nblk + i, 0)),
        out_shape=jax.ShapeDtypeStruct((r * n, do), jnp.float32),
    )(p, Wr)


def _add_body(q_ref, o_ref, *, d_out):
    o_ref[...] = (q_ref[0] + q_ref[1])[:, :d_out]


def _partial_add(q, bn, d_out):
    _, n, d = q.shape
    return pl.pallas_call(
        functools.partial(_add_body, d_out=d_out),
        grid=(n // bn,),
        in_specs=[pl.BlockSpec((2, bn, d), lambda i: (0, i, 0))],
        out_specs=pl.BlockSpec((bn, d_out), lambda i: (i, 0)),
        out_shape=jax.ShapeDtypeStruct((n, d_out), jnp.float32),
    )(q)


def _sc_aggregate(table, idx3, dst3, n_nodes):
    d = table.shape[1]
    tiles, nb, b = idx3.shape
    chunks = -(-n_nodes // (b * _NS))
    acc_rows = chunks * b * _NS
    rpt = ((n_nodes // _NS + 7) // 8) * 8
    rlast = n_nodes - rpt * (_NS - 1)
    assert rlast > 0 and rlast % 8 == 0
    assert nb % 4 == 0

    @functools.partial(
        pl.kernel,
        out_type=jax.ShapeDtypeStruct((_NC, n_nodes, d), jnp.float32),
        mesh=plsc.VectorSubcoreMesh(core_axis_name="c", subcore_axis_name="s"),
        scratch_types=[
            pltpu.VMEM((nb, b), jnp.int32),
            pltpu.VMEM((nb, b), jnp.int32),
            pltpu.VMEM((b, d), jnp.float32),
            pltpu.VMEM((b, d), jnp.float32),
            pltpu.VMEM((b, d), jnp.float32),
            pltpu.VMEM((b, d), jnp.float32),
            pltpu.VMEM_SHARED((acc_rows, d), jnp.float32),
            pltpu.SemaphoreType.DMA,
            pltpu.SemaphoreType.DMA,
            pltpu.SemaphoreType.DMA,
            pltpu.SemaphoreType.DMA,
        ],
        compiler_params=pltpu.CompilerParams(use_tc_tiling_on_sc=False),
    )
    def sc_k(table_h, idx_h, dst_h, out_h, idx_v, dst_v, b0, b1, b2, b3, acc,
             s0, s1, s2, s3):
        bufs = (b0, b1, b2, b3)
        sems = (s0, s1, s2, s3)
        c = lax.axis_index("c")
        s = lax.axis_index("s")
        w = c * _NS + s

        pltpu.sync_copy(idx_h.at[w], idx_v)
        pltpu.sync_copy(dst_h.at[w], dst_v)

        zvec = jnp.zeros((16,), jnp.float32)

        def _zrow(i, carry):
            for k2 in range(d // 16):
                b0[i, pl.ds(k2 * 16, 16)] = zvec
            return carry

        lax.fori_loop(0, b, _zrow, 0)

        def _zdma(i, carry):
            pltpu.sync_copy(b0, acc.at[pl.ds(pl.multiple_of((s * chunks + i) * b, 8), b)])
            return carry

        lax.fori_loop(0, chunks, _zdma, 0)
        plsc.subcore_barrier()

        for t in range(3):
            pltpu.async_copy(table_h.at[idx_v.at[t]], bufs[t], sems[t])

        def _grp(g, carry):
            j0 = g * 4
            for t in range(4):
                j = j0 + t
                tp = (t + 3) % 4

                @pl.when(j + 3 < nb)
                def _():
                    pltpu.async_copy(table_h.at[idx_v.at[j + 3]], bufs[tp], sems[tp])

                pltpu.make_async_copy(table_h.at[idx_v.at[j]], bufs[t], sems[t]).wait()
                pltpu.sync_copy(bufs[t], acc.at[dst_v.at[j]], add=True)
            return carry

        lax.fori_loop(0, nb // 4, _grp, 0)

        plsc.subcore_barrier()
        base = pl.multiple_of(s * rpt, 8)

        @pl.when(s < _NS - 1)
        def _():
            pltpu.sync_copy(acc.at[pl.ds(base, rpt)], out_h.at[c].at[pl.ds(base, rpt)])

        @pl.when(s == _NS - 1)
        def _():
            lbase = (_NS - 1) * rpt
            pltpu.sync_copy(acc.at[pl.ds(lbase, rlast)], out_h.at[c].at[pl.ds(lbase, rlast)])

    return sc_k(table, idx3, dst3)


def kernel(x, edge_index, edge_type, W1, W2):
    n, d_in = x.shape
    e = edge_type.shape[0]
    d_hid = W1.shape[1]
    d_out = W2.shape[1]
    r = W1.shape[0] // d_in

    src = edge_index[0]
    dst = edge_index[1]
    W1r = W1.reshape(r, d_in, d_hid)
    d2 = 48
    W2r = jnp.pad(W2.reshape(r, d_hid, d_out), ((0, 0), (0, 0), (0, d2 - d_out)))

    tiles = _NC * _NS
    epw = e // tiles
    nb = -(-epw // _B)
    nb = ((nb + 3) // 4) * 4
    pad = nb * _B - epw
    idx3 = jnp.pad(_flat_index(src, edge_type, n).reshape(tiles, epw),
                   ((0, 0), (0, pad))).reshape(tiles, nb, _B)
    dst_pad = jnp.broadcast_to(n + jnp.arange(pad, dtype=dst.dtype), (tiles, pad))
    dst3 = jnp.concatenate([dst.reshape(tiles, epw), dst_pad],
                           axis=1).reshape(tiles, nb, _B)

    y1 = _rel_transform(x, W1r, 2000)
    p1 = _sc_aggregate(y1, idx3, dst3, n)
    y2 = _rel_transform_fused(p1, W2r, 2000)
    p2 = _sc_aggregate(y2, idx3, dst3, n)
    return _partial_add(p2, 1000, d_out)


# speedup vs baseline: 1.8034x; 1.8034x over previous
import functools

import jax
import jax.numpy as jnp
from jax import lax
from jax.experimental import pallas as pl
from jax.experimental.pallas import tpu as pltpu
from jax.experimental.pallas import tpu_sc as plsc

_NC = 2
_NS = 16
_B = 80


def _idx_body(src_ref, et_ref, o_ref, *, n_nodes):
    o_ref[...] = et_ref[...] * n_nodes + src_ref[...]


def _flat_index(src, et, n_nodes):
    e = src.shape[0]
    src2 = src.reshape(e // 128, 128)
    et2 = et.reshape(e // 128, 128)
    rows = e // 128
    out = pl.pallas_call(
        functools.partial(_idx_body, n_nodes=n_nodes),
        out_shape=jax.ShapeDtypeStruct((rows, 128), jnp.int32),
    )(src2, et2)
    return out.reshape(-1)


def _mm_plain_body(x_ref, w_ref, y_ref):
    y_ref[...] = jnp.dot(x_ref[...], w_ref[0], preferred_element_type=jnp.float32)


def _rel_transform(x, Wr, bn):
    n, d = x.shape
    r, _, do = Wr.shape
    nblk = n // bn
    return pl.pallas_call(
        _mm_plain_body,
        grid=(nblk, r),
        in_specs=[
            pl.BlockSpec((bn, d), lambda i, j: (i, 0)),
            pl.BlockSpec((1, d, do), lambda i, j: (j, 0, 0)),
        ],
        out_specs=pl.BlockSpec((bn, do), lambda i, j: (j * nblk + i, 0)),
        out_shape=jax.ShapeDtypeStruct((r * n, do), jnp.float32),
    )(x, Wr)


def _mm_fused_body(p_ref, w_ref, y_ref):
    h = jnp.maximum(p_ref[0] + p_ref[1], 0.0)
    y_ref[...] = jnp.dot(h, w_ref[0], preferred_element_type=jnp.float32)


def _rel_transform_fused(p, Wr, bn):
    _, n, d = p.shape
    r, _, do = Wr.shape
    nblk = n // bn
    return pl.pallas_call(
        _mm_fused_body,
        grid=(nblk, r),
        in_specs=[
            pl.BlockSpec((2, bn, d), lambda i, j: (0, i, 0)),
            pl.BlockSpec((1, d, do), lambda i, j: (j, 0, 0)),
        ],
        out_specs=pl.BlockSpec((bn, do), lambda i, j: (j * nblk + i, 0)),
        out_shape=jax.ShapeDtypeStruct((r * n, do), jnp.float32),
    )(p, Wr)


def _add_body(q_ref, o_ref, *, d_out):
    o_ref[...] = (q_ref[0] + q_ref[1])[:, :d_out]


def _partial_add(q, bn, d_out):
    _, n, d = q.shape
    return pl.pallas_call(
        functools.partial(_add_body, d_out=d_out),
        grid=(n // bn,),
        in_specs=[pl.BlockSpec((2, bn, d), lambda i: (0, i, 0))],
        out_specs=pl.BlockSpec((bn, d_out), lambda i: (i, 0)),
        out_shape=jax.ShapeDtypeStruct((n, d_out), jnp.float32),
    )(q)


def _sc_aggregate(table, idx3, dst3, n_nodes):
    d = table.shape[1]
    tiles, nb, b = idx3.shape
    chunks = -(-n_nodes // (b * _NS))
    acc_rows = chunks * b * _NS
    rpt = ((n_nodes // _NS + 7) // 8) * 8
    rlast = n_nodes - rpt * (_NS - 1)
    assert rlast > 0 and rlast % 8 == 0
    assert nb % 4 == 0

    @functools.partial(
        pl.kernel,
        out_type=jax.ShapeDtypeStruct((_NC, n_nodes, d), jnp.float32),
        mesh=plsc.VectorSubcoreMesh(core_axis_name="c", subcore_axis_name="s"),
        scratch_types=[
            pltpu.VMEM((nb, b), jnp.int32),
            pltpu.VMEM((nb, b), jnp.int32),
            pltpu.VMEM((b, d), jnp.float32),
            pltpu.VMEM((b, d), jnp.float32),
            pltpu.VMEM((b, d), jnp.float32),
            pltpu.VMEM((b, d), jnp.float32),
            pltpu.VMEM_SHARED((acc_rows, d), jnp.float32),
            pltpu.SemaphoreType.DMA,
            pltpu.SemaphoreType.DMA,
            pltpu.SemaphoreType.DMA,
            pltpu.SemaphoreType.DMA,
        ],
        compiler_params=pltpu.CompilerParams(use_tc_tiling_on_sc=False),
    )
    def sc_k(table_h, idx_h, dst_h, out_h, idx_v, dst_v, b0, b1, b2, b3, acc,
             s0, s1, s2, s3):
        bufs = (b0, b1, b2, b3)
        sems = (s0, s1, s2, s3)
        c = lax.axis_index("c")
        s = lax.axis_index("s")
        w = c * _NS + s

        pltpu.sync_copy(idx_h.at[w], idx_v)
        pltpu.sync_copy(dst_h.at[w], dst_v)

        zvec = jnp.zeros((16,), jnp.float32)

        def _zrow(i, carry):
            for k2 in range(d // 16):
                b0[i, pl.ds(k2 * 16, 16)] = zvec
            return carry

        lax.fori_loop(0, b, _zrow, 0)

        def _zdma(i, carry):
            pltpu.sync_copy(b0, acc.at[pl.ds(pl.multiple_of((s * chunks + i) * b, 8), b)])
            return carry

        lax.fori_loop(0, chunks, _zdma, 0)
        plsc.subcore_barrier()

        for t in range(3):
            pltpu.async_copy(table_h.at[idx_v.at[t]], bufs[t], sems[t])

        def _grp(g, carry):
            j0 = g * 4
            for t in range(4):
                j = j0 + t
                tp = (t + 3) % 4

                @pl.when(j + 3 < nb)
                def _():
                    pltpu.async_copy(table_h.at[idx_v.at[j + 3]], bufs[tp], sems[tp])

                pltpu.make_async_copy(table_h.at[idx_v.at[j]], bufs[t], sems[t]).wait()
                pltpu.sync_copy(bufs[t], acc.at[dst_v.at[j]], add=True)
            return carry

        lax.fori_loop(0, nb // 4, _grp, 0)

        plsc.subcore_barrier()
        base = pl.multiple_of(s * rpt, 8)

        @pl.when(s < _NS - 1)
        def _():
            pltpu.sync_copy(acc.at[pl.ds(base, rpt)], out_h.at[c].at[pl.ds(base, rpt)])

        @pl.when(s == _NS - 1)
        def _():
            lbase = (_NS - 1) * rpt
            pltpu.sync_copy(acc.at[pl.ds(lbase, rlast)], out_h.at[c].at[pl.ds(lbase, rlast)])

    return sc_k(table, idx3, dst3)


def kernel(x, edge_index, edge_type, W1, W2):
    n, d_in = x.shape
    e = edge_type.shape[0]
    d_hid = W1.shape[1]
    d_out = W2.shape[1]
    r = W1.shape[0] // d_in

    src = edge_index[0]
    dst = edge_index[1]
    W1r = W1.reshape(r, d_in, d_hid)
    d2 = 48
    W2r = jnp.pad(W2.reshape(r, d_hid, d_out), ((0, 0), (0, 0), (0, d2 - d_out)))

    tiles = _NC * _NS
    epw = e // tiles
    nb = -(-epw // _B)
    nb = ((nb + 3) // 4) * 4
    pad = nb * _B - epw
    idx_pad = jnp.broadcast_to(jnp.arange(pad, dtype=jnp.int32), (tiles, pad))
    idx3 = jnp.concatenate([_flat_index(src, edge_type, n).reshape(tiles, epw),
                            idx_pad], axis=1).reshape(tiles, nb, _B)
    dst_pad = jnp.broadcast_to(n + jnp.arange(pad, dtype=dst.dtype), (tiles, pad))
    dst3 = jnp.concatenate([dst.reshape(tiles, epw), dst_pad],
                           axis=1).reshape(tiles, nb, _B)

    y1 = _rel_transform(x, W1r, 2000)
    p1 = _sc_aggregate(y1, idx3, dst3, n)
    y2 = _rel_transform_fused(p1, W2r, 2000)
    p2 = _sc_aggregate(y2, idx3, dst3, n)
    return _partial_add(p2, 1000, d_out)


# speedup vs baseline: 2.0707x; 1.1482x over previous
import functools

import jax
import jax.numpy as jnp
from jax import lax
from jax.experimental import pallas as pl
from jax.experimental.pallas import tpu as pltpu
from jax.experimental.pallas import tpu_sc as plsc

_NC = 2
_NS = 16
_B = 80


def _idx_body(src_ref, et_ref, o_ref, *, n_nodes):
    o_ref[...] = et_ref[...] * n_nodes + src_ref[...]


def _flat_index(src, et, n_nodes):
    e = src.shape[0]
    src2 = src.reshape(e // 128, 128)
    et2 = et.reshape(e // 128, 128)
    rows = e // 128
    out = pl.pallas_call(
        functools.partial(_idx_body, n_nodes=n_nodes),
        out_shape=jax.ShapeDtypeStruct((rows, 128), jnp.int32),
    )(src2, et2)
    return out.reshape(-1)


def _mm_plain_body(x_ref, w_ref, y_ref):
    y_ref[...] = jnp.dot(x_ref[...], w_ref[0], preferred_element_type=jnp.float32)


def _rel_transform(x, Wr, bn):
    n, d = x.shape
    r, _, do = Wr.shape
    nblk = n // bn
    return pl.pallas_call(
        _mm_plain_body,
        grid=(nblk, r),
        in_specs=[
            pl.BlockSpec((bn, d), lambda i, j: (i, 0)),
            pl.BlockSpec((1, d, do), lambda i, j: (j, 0, 0)),
        ],
        out_specs=pl.BlockSpec((bn, do), lambda i, j: (j * nblk + i, 0)),
        out_shape=jax.ShapeDtypeStruct((r * n, do), jnp.float32),
    )(x, Wr)


def _mm_fused_body(p_ref, w_ref, y_ref):
    h = jnp.maximum(p_ref[0] + p_ref[1], 0.0)
    y_ref[...] = jnp.dot(h, w_ref[0], preferred_element_type=jnp.float32)


def _rel_transform_fused(p, Wr, bn):
    _, n, d = p.shape
    r, _, do = Wr.shape
    nblk = n // bn
    return pl.pallas_call(
        _mm_fused_body,
        grid=(nblk, r),
        in_specs=[
            pl.BlockSpec((2, bn, d), lambda i, j: (0, i, 0)),
            pl.BlockSpec((1, d, do), lambda i, j: (j, 0, 0)),
        ],
        out_specs=pl.BlockSpec((bn, do), lambda i, j: (j * nblk + i, 0)),
        out_shape=jax.ShapeDtypeStruct((r * n, do), jnp.float32),
    )(p, Wr)


def _add_body(q_ref, o_ref, *, d_out):
    o_ref[...] = (q_ref[0] + q_ref[1])[:, :d_out]


def _partial_add(q, bn, d_out):
    _, n, d = q.shape
    return pl.pallas_call(
        functools.partial(_add_body, d_out=d_out),
        grid=(n // bn,),
        in_specs=[pl.BlockSpec((2, bn, d), lambda i: (0, i, 0))],
        out_specs=pl.BlockSpec((bn, d_out), lambda i: (i, 0)),
        out_shape=jax.ShapeDtypeStruct((n, d_out), jnp.float32),
    )(q)


def _sc_aggregate(table, idx3, dst3, n_nodes):
    d = table.shape[1]
    tiles, nb, b = idx3.shape
    chunks = -(-n_nodes // (b * _NS))
    acc_rows = chunks * b * _NS
    rpt = ((n_nodes // _NS + 7) // 8) * 8
    rlast = n_nodes - rpt * (_NS - 1)
    assert rlast > 0 and rlast % 8 == 0
    assert nb % 4 == 0

    @functools.partial(
        pl.kernel,
        out_type=jax.ShapeDtypeStruct((_NC, n_nodes, d), jnp.float32),
        mesh=plsc.VectorSubcoreMesh(core_axis_name="c", subcore_axis_name="s"),
        scratch_types=[
            pltpu.VMEM((nb, b), jnp.int32),
            pltpu.VMEM((nb, b), jnp.int32),
            pltpu.VMEM((b, d), jnp.float32),
            pltpu.VMEM((b, d), jnp.float32),
            pltpu.VMEM((b, d), jnp.float32),
            pltpu.VMEM((b, d), jnp.float32),
            pltpu.VMEM_SHARED((acc_rows, d), jnp.float32),
            pltpu.SemaphoreType.DMA,
            pltpu.SemaphoreType.DMA,
            pltpu.SemaphoreType.DMA,
            pltpu.SemaphoreType.DMA,
        ],
        compiler_params=pltpu.CompilerParams(use_tc_tiling_on_sc=False),
    )
    def sc_k(table_h, idx_h, dst_h, out_h, idx_v, dst_v, b0, b1, b2, b3, acc,
             s0, s1, s2, s3):
        bufs = (b0, b1, b2, b3)
        sems = (s0, s1, s2, s3)
        c = lax.axis_index("c")
        s = lax.axis_index("s")
        w = c * _NS + s

        pltpu.sync_copy(idx_h.at[w], idx_v)
        pltpu.sync_copy(dst_h.at[w], dst_v)

        zvec = jnp.zeros((16,), jnp.float32)

        def _zrow(i, carry):
            for k2 in range(d // 16):
                b0[i, pl.ds(k2 * 16, 16)] = zvec
            return carry

        lax.fori_loop(0, b, _zrow, 0)

        def _zdma(i, carry):
            pltpu.sync_copy(b0, acc.at[pl.ds(pl.multiple_of((s * chunks + i) * b, 8), b)])
            return carry

        lax.fori_loop(0, chunks, _zdma, 0)
        plsc.subcore_barrier()

        for t in range(3):
            pltpu.async_copy(table_h.at[idx_v.at[t]], bufs[t], sems[t])

        def _grp(g, carry):
            j0 = g * 4
            for t in range(4):
                j = j0 + t
                tp = (t + 3) % 4

                @pl.when(j + 3 < nb)
                def _():
                    pltpu.async_copy(table_h.at[idx_v.at[j + 3]], bufs[tp], sems[tp])

                pltpu.make_async_copy(table_h.at[idx_v.at[j]], bufs[t], sems[t]).wait()
                pltpu.sync_copy(bufs[t], acc.at[dst_v.at[j]], add=True)
            return carry

        lax.fori_loop(0, nb // 4, _grp, 0)

        plsc.subcore_barrier()
        base = pl.multiple_of(s * rpt, 8)

        @pl.when(s < _NS - 1)
        def _():
            pltpu.sync_copy(acc.at[pl.ds(base, rpt)], out_h.at[c].at[pl.ds(base, rpt)])

        @pl.when(s == _NS - 1)
        def _():
            lbase = (_NS - 1) * rpt
            pltpu.sync_copy(acc.at[pl.ds(lbase, rlast)], out_h.at[c].at[pl.ds(lbase, rlast)])

    return sc_k(table, idx3, dst3)


def kernel(x, edge_index, edge_type, W1, W2):
    n, d_in = x.shape
    e = edge_type.shape[0]
    d_hid = W1.shape[1]
    d_out = W2.shape[1]
    r = W1.shape[0] // d_in

    src = edge_index[0]
    dst = edge_index[1]
    W1r = W1.reshape(r, d_in, d_hid)
    d2 = 48
    W2r = jnp.pad(W2.reshape(r, d_hid, d_out), ((0, 0), (0, 0), (0, d2 - d_out)))

    tiles = _NC * _NS
    epw = e // tiles
    nb = -(-epw // _B)
    nb = ((nb + 3) // 4) * 4
    pad = nb * _B - epw
    idx_pad = jnp.broadcast_to(jnp.arange(pad, dtype=jnp.int32), (tiles, pad))
    idx3 = jnp.concatenate([_flat_index(src, edge_type, n).reshape(tiles, epw),
                            idx_pad], axis=1).reshape(tiles, nb, _B)
    dst_pad = jnp.broadcast_to(n + jnp.arange(pad, dtype=dst.dtype), (tiles, pad))
    dst3 = jnp.concatenate([dst.reshape(tiles, epw), dst_pad],
                           axis=1).reshape(tiles, nb, _B)

    y1 = _rel_transform(x, W1r, n)
    p1 = _sc_aggregate(y1, idx3, dst3, n)
    y2 = _rel_transform_fused(p1, W2r, n)
    p2 = _sc_aggregate(y2, idx3, dst3, n)
    return _partial_add(p2, n, d_out)


# speedup vs baseline: 2.1067x; 1.0174x over previous
import functools

import jax
import jax.numpy as jnp
from jax import lax
from jax.experimental import pallas as pl
from jax.experimental.pallas import tpu as pltpu
from jax.experimental.pallas import tpu_sc as plsc

_NC = 2
_NS = 16
_B = 80


def _idx_body(ei_ref, et_ref, o_ref, *, n_nodes):
    o_ref[...] = et_ref[...] * n_nodes + ei_ref[0]


def _flat_index(edge_index, edge_type, n_nodes):
    e = edge_type.shape[0]
    return pl.pallas_call(
        functools.partial(_idx_body, n_nodes=n_nodes),
        out_shape=jax.ShapeDtypeStruct((e,), jnp.int32),
    )(edge_index, edge_type)


def _mm_plain_body(x_ref, w_ref, y_ref):
    y_ref[...] = jnp.dot(x_ref[...], w_ref[0], preferred_element_type=jnp.float32)


def _rel_transform(x, Wr, bn):
    n, d = x.shape
    r, _, do = Wr.shape
    nblk = n // bn
    return pl.pallas_call(
        _mm_plain_body,
        grid=(nblk, r),
        in_specs=[
            pl.BlockSpec((bn, d), lambda i, j: (i, 0)),
            pl.BlockSpec((1, d, do), lambda i, j: (j, 0, 0)),
        ],
        out_specs=pl.BlockSpec((bn, do), lambda i, j: (j * nblk + i, 0)),
        out_shape=jax.ShapeDtypeStruct((r * n, do), jnp.float32),
    )(x, Wr)


def _mm_fused_body(p_ref, w_ref, y_ref):
    h = jnp.maximum(p_ref[0] + p_ref[1], 0.0)
    y_ref[...] = jnp.dot(h, w_ref[0], preferred_element_type=jnp.float32)


def _rel_transform_fused(p, Wr, bn):
    _, n, d = p.shape
    r, _, do = Wr.shape
    nblk = n // bn
    return pl.pallas_call(
        _mm_fused_body,
        grid=(nblk, r),
        in_specs=[
            pl.BlockSpec((2, bn, d), lambda i, j: (0, i, 0)),
            pl.BlockSpec((1, d, do), lambda i, j: (j, 0, 0)),
        ],
        out_specs=pl.BlockSpec((bn, do), lambda i, j: (j * nblk + i, 0)),
        out_shape=jax.ShapeDtypeStruct((r * n, do), jnp.float32),
    )(p, Wr)


def _add_body(q_ref, o_ref, *, d_out):
    o_ref[...] = (q_ref[0] + q_ref[1])[:, :d_out]


def _partial_add(q, bn, d_out):
    _, n, d = q.shape
    return pl.pallas_call(
        functools.partial(_add_body, d_out=d_out),
        grid=(n // bn,),
        in_specs=[pl.BlockSpec((2, bn, d), lambda i: (0, i, 0))],
        out_specs=pl.BlockSpec((bn, d_out), lambda i: (i, 0)),
        out_shape=jax.ShapeDtypeStruct((n, d_out), jnp.float32),
    )(q)


def _sc_aggregate(table, idx3, dst3, n_nodes):
    d = table.shape[1]
    tiles, nb_h, b = idx3.shape
    nb = ((nb_h + 3) // 4) * 4
    chunks = -(-n_nodes // (b * _NS))
    acc_rows = chunks * b * _NS
    assert n_nodes + (nb - nb_h) * b <= acc_rows
    rpt = ((n_nodes // _NS + 7) // 8) * 8
    rlast = n_nodes - rpt * (_NS - 1)
    assert rlast > 0 and rlast % 8 == 0

    @functools.partial(
        pl.kernel,
        out_type=jax.ShapeDtypeStruct((_NC, n_nodes, d), jnp.float32),
        mesh=plsc.VectorSubcoreMesh(core_axis_name="c", subcore_axis_name="s"),
        scratch_types=[
            pltpu.VMEM((nb, b), jnp.int32),
            pltpu.VMEM((nb, b), jnp.int32),
            pltpu.VMEM((b, d), jnp.float32),
            pltpu.VMEM((b, d), jnp.float32),
            pltpu.VMEM((b, d), jnp.float32),
            pltpu.VMEM((b, d), jnp.float32),
            pltpu.VMEM_SHARED((acc_rows, d), jnp.float32),
            pltpu.SemaphoreType.DMA,
            pltpu.SemaphoreType.DMA,
            pltpu.SemaphoreType.DMA,
            pltpu.SemaphoreType.DMA,
        ],
        compiler_params=pltpu.CompilerParams(use_tc_tiling_on_sc=False),
    )
    def sc_k(table_h, idx_h, dst_h, out_h, idx_v, dst_v, b0, b1, b2, b3, acc,
             s0, s1, s2, s3):
        bufs = (b0, b1, b2, b3)
        sems = (s0, s1, s2, s3)
        c = lax.axis_index("c")
        s = lax.axis_index("s")
        w = c * _NS + s

        pltpu.sync_copy(idx_h.at[w], idx_v.at[pl.ds(0, nb_h)])
        pltpu.sync_copy(dst_h.at[w], dst_v.at[pl.ds(0, nb_h)])
        lane = lax.iota(jnp.int32, 16)
        for j in range(nb_h, nb):
            for i in range(b // 16):
                off = (j - nb_h) * b + i * 16
                idx_v[j, pl.ds(i * 16, 16)] = off + lane
                dst_v[j, pl.ds(i * 16, 16)] = n_nodes + off + lane

        zvec = jnp.zeros((16,), jnp.float32)

        def _zrow(i, carry):
            for k2 in range(d // 16):
                b0[i, pl.ds(k2 * 16, 16)] = zvec
            return carry

        lax.fori_loop(0, b, _zrow, 0)

        def _zdma(i, carry):
            pltpu.sync_copy(b0, acc.at[pl.ds(pl.multiple_of((s * chunks + i) * b, 8), b)])
            return carry

        lax.fori_loop(0, chunks, _zdma, 0)
        plsc.subcore_barrier()

        for t in range(3):
            pltpu.async_copy(table_h.at[idx_v.at[t]], bufs[t], sems[t])

        def _grp(g, carry):
            j0 = g * 4
            for t in range(4):
                j = j0 + t
                tp = (t + 3) % 4

                @pl.when(j + 3 < nb)
                def _():
                    pltpu.async_copy(table_h.at[idx_v.at[j + 3]], bufs[tp], sems[tp])

                pltpu.make_async_copy(table_h.at[idx_v.at[j]], bufs[t], sems[t]).wait()
                pltpu.sync_copy(bufs[t], acc.at[dst_v.at[j]], add=True)
            return carry

        lax.fori_loop(0, nb // 4, _grp, 0)

        plsc.subcore_barrier()
        base = pl.multiple_of(s * rpt, 8)

        @pl.when(s < _NS - 1)
        def _():
            pltpu.sync_copy(acc.at[pl.ds(base, rpt)], out_h.at[c].at[pl.ds(base, rpt)])

        @pl.when(s == _NS - 1)
        def _():
            lbase = (_NS - 1) * rpt
            pltpu.sync_copy(acc.at[pl.ds(lbase, rlast)], out_h.at[c].at[pl.ds(lbase, rlast)])

    return sc_k(table, idx3, dst3)


def kernel(x, edge_index, edge_type, W1, W2):
    n, d_in = x.shape
    e = edge_type.shape[0]
    d_hid = W1.shape[1]
    d_out = W2.shape[1]
    r = W1.shape[0] // d_in

    src = edge_index[0]
    dst = edge_index[1]
    W1r = W1.reshape(r, d_in, d_hid)
    d2 = 48
    W2r = jnp.pad(W2.reshape(r, d_hid, d_out), ((0, 0), (0, 0), (0, d2 - d_out)))

    tiles = _NC * _NS
    epw = e // tiles
    assert epw % _B == 0
    idx3 = _flat_index(edge_index, edge_type, n).reshape(tiles, epw // _B, _B)
    dst3 = dst.reshape(tiles, epw // _B, _B)

    y1 = _rel_transform(x, W1r, n)
    p1 = _sc_aggregate(y1, idx3, dst3, n)
    y2 = _rel_transform_fused(p1, W2r, n)
    p2 = _sc_aggregate(y2, idx3, dst3, n)
    return _partial_add(p2, n, d_out)
